# Initial kernel scaffold; baseline (speedup 1.0000x reference)
import jax
import jax.numpy as jnp
from jax.experimental import pallas as pl


def kernel(hidden_states, position_ids, ln1_w, wq, wk, wv, wo, sinks, ln2_w, router_w, router_b, w_gate, b_gate, w_up, b_up, w_down, b_down):
    raise NotImplementedError("write your pallas kernel here")


# speedup vs baseline: 1.3825x; 1.3825x over previous
import jax
import jax.numpy as jnp
from jax.experimental import pallas as pl
from jax.experimental.pallas import tpu as pltpu

B, S, D = 1, 2048, 1024
H, KVH, HD = 16, 8, 64
E, F = 8, 1024
EPS = 1e-5
BASE = 10000.0
NEG = -1e9
BS = 256


def _qkv_body(pos_ref, invf_ref, x_ref, ln1_ref, wq_ref, wqr_ref, wk_ref,
              wkr_ref, wv_ref, q_ref, k_ref, v_ref):
    x = x_ref[...]
    var = jnp.mean(jnp.square(x), axis=1, keepdims=True)
    r = (x * jax.lax.rsqrt(var + EPS) * ln1_ref[...]).astype(jnp.bfloat16)
    ang = pos_ref[...] * invf_ref[...]
    c = jnp.cos(ang)
    s = jnp.sin(ang)
    q0 = jnp.dot(r, wq_ref[...], preferred_element_type=jnp.float32)
    q1 = jnp.dot(r, wqr_ref[...], preferred_element_type=jnp.float32)
    q_ref[...] = (q0 * c + q1 * s).astype(jnp.bfloat16)
    k0 = jnp.dot(r, wk_ref[...], preferred_element_type=jnp.float32)
    k1 = jnp.dot(r, wkr_ref[...], preferred_element_type=jnp.float32)
    kv = KVH * HD
    k_ref[...] = (k0 * c[:, :kv] + k1 * s[:, :kv]).astype(jnp.bfloat16)
    v_ref[...] = jnp.dot(r, wv_ref[...],
                         preferred_element_type=jnp.float32).astype(jnp.bfloat16)


def _attn_body(sinks_ref, q_ref, k_ref, v_ref, o_ref):
    h = pl.program_id(0)
    i = pl.program_id(1)
    q = q_ref[0]
    k = k_ref[0]
    sc = jax.lax.dot_general(q, k, (((1,), (1,)), ((), ())),
                             preferred_element_type=jnp.float32) * 0.125
    row = jax.lax.broadcasted_iota(jnp.int32, (BS, S), 0) + i * BS
    col = jax.lax.broadcasted_iota(jnp.int32, (BS, S), 1)
    sc = jnp.where(col <= row, sc, NEG)
    sink = sinks_ref[h]
    m = jnp.maximum(jnp.max(sc, axis=1, keepdims=True), sink)
    p = jnp.exp(sc - m)
    l = jnp.sum(p, axis=1, keepdims=True) + jnp.exp(sink - m)
    o = jnp.dot(p.astype(jnp.bfloat16), v_ref[0],
                preferred_element_type=jnp.float32) / l
    o_ref[0] = o.astype(jnp.bfloat16)


def _post_body(x_ref, a_ref, wo_ref, ln2_ref, rw_ref, rb_ref,
               x2_ref, t_ref, i0_ref, i1_ref, w0_ref, w1_ref):
    x2 = x_ref[...] + jnp.dot(a_ref[...], wo_ref[...],
                              preferred_element_type=jnp.float32)
    x2_ref[...] = x2
    var = jnp.mean(jnp.square(x2), axis=1, keepdims=True)
    t = x2 * jax.lax.rsqrt(var + EPS) * ln2_ref[...]
    t_ref[...] = t.astype(jnp.bfloat16)
    logits = jnp.dot(t, rw_ref[...],
                     preferred_element_type=jnp.float32) + rb_ref[...]
    iota = jax.lax.broadcasted_iota(jnp.int32, (BS, E), 1)
    m0 = jnp.max(logits, axis=1, keepdims=True)
    i0 = jnp.min(jnp.where(logits == m0, iota, E), axis=1, keepdims=True)
    l2 = jnp.where(iota == i0, -1e30, logits)
    m1 = jnp.max(l2, axis=1, keepdims=True)
    i1 = jnp.min(jnp.where(l2 == m1, iota, E), axis=1, keepdims=True)
    w0 = jax.nn.sigmoid(m0 - m1)
    i0_ref[...] = i0
    i1_ref[...] = i1
    w0_ref[...] = w0
    w1_ref[...] = 1.0 - w0


def _moe_body(x2_ref, t_ref, i0_ref, i1_ref, w0_ref, w1_ref,
              wg_ref, bg_ref, wu_ref, bu_ref, wd_ref, bd_ref, o_ref):
    e = pl.program_id(1)
    t = t_ref[...]
    gate = jnp.dot(t, wg_ref[0], preferred_element_type=jnp.float32) + bg_ref[0]
    up = jnp.dot(t, wu_ref[0], preferred_element_type=jnp.float32) + bu_ref[0]
    gate = jnp.minimum(gate, 7.0)
    up = jnp.clip(up, -7.0, 7.0)
    act = (up + 1.0) * (gate * jax.nn.sigmoid(1.702 * gate))
    y = jnp.dot(act.astype(jnp.bfloat16), wd_ref[0],
                preferred_element_type=jnp.float32) + bd_ref[0]
    w_e = (jnp.where(i0_ref[...] == e, w0_ref[...], 0.0) +
           jnp.where(i1_ref[...] == e, w1_ref[...], 0.0))
    contrib = w_e * y

    @pl.when(e == 0)
    def _():
        o_ref[...] = x2_ref[...] + contrib

    @pl.when(e > 0)
    def _():
        o_ref[...] += contrib


def _rot_cols(w, nh):
    w4 = w.reshape(D, nh, 2, HD // 2)
    return jnp.concatenate([-w4[:, :, 1:2, :], w4[:, :, 0:1, :]],
                           axis=2).reshape(D, nh * HD)


def kernel(hidden_states, position_ids, ln1_w, wq, wk, wv, wo, sinks, ln2_w,
           router_w, router_b, w_gate, b_gate, w_up, b_up, w_down, b_down):
    x = hidden_states.reshape(S, D)
    pos = position_ids.reshape(S, 1).astype(jnp.float32)
    inv_freq = 1.0 / (BASE ** (jnp.arange(0, HD, 2, dtype=jnp.float32) / HD))
    invf = jnp.tile(jnp.concatenate([inv_freq, inv_freq]), H).reshape(1, D)

    bf = jnp.bfloat16
    wq_b, wk_b, wv_b = wq.astype(bf), wk.astype(bf), wv.astype(bf)
    wqr_b = _rot_cols(wq, H).astype(bf)
    wkr_b = _rot_cols(wk, KVH).astype(bf)
    wo_b = wo.astype(bf)
    wg_b, wu_b, wd_b = w_gate.astype(bf), w_up.astype(bf), w_down.astype(bf)

    nS = S // BS
    kv = KVH * HD

    q, k, v = pl.pallas_call(
        _qkv_body,
        grid=(nS,),
        in_specs=[
            pl.BlockSpec((BS, 1), lambda i: (i, 0)),
            pl.BlockSpec((1, D), lambda i: (0, 0)),
            pl.BlockSpec((BS, D), lambda i: (i, 0)),
            pl.BlockSpec((1, D), lambda i: (0, 0)),
            pl.BlockSpec((D, D), lambda i: (0, 0)),
            pl.BlockSpec((D, D), lambda i: (0, 0)),
            pl.BlockSpec((D, kv), lambda i: (0, 0)),
            pl.BlockSpec((D, kv), lambda i: (0, 0)),
            pl.BlockSpec((D, kv), lambda i: (0, 0)),
        ],
        out_specs=[
            pl.BlockSpec((BS, D), lambda i: (i, 0)),
            pl.BlockSpec((BS, kv), lambda i: (i, 0)),
            pl.BlockSpec((BS, kv), lambda i: (i, 0)),
        ],
        out_shape=[
            jax.ShapeDtypeStruct((S, D), bf),
            jax.ShapeDtypeStruct((S, kv), bf),
            jax.ShapeDtypeStruct((S, kv), bf),
        ],
    )(pos, invf, x, ln1_w.reshape(1, D), wq_b, wqr_b, wk_b, wkr_b, wv_b)

    qh = q.reshape(S, H, HD).transpose(1, 0, 2)
    kh = k.reshape(S, KVH, HD).transpose(1, 0, 2)
    vh = v.reshape(S, KVH, HD).transpose(1, 0, 2)
    oh = pl.pallas_call(
        _attn_body,
        grid=(H, nS),
        in_specs=[
            pl.BlockSpec(memory_space=pltpu.SMEM),
            pl.BlockSpec((1, BS, HD), lambda h, i: (h, i, 0)),
            pl.BlockSpec((1, S, HD), lambda h, i: (h // 2, 0, 0)),
            pl.BlockSpec((1, S, HD), lambda h, i: (h // 2, 0, 0)),
        ],
        out_specs=pl.BlockSpec((1, BS, HD), lambda h, i: (h, i, 0)),
        out_shape=jax.ShapeDtypeStruct((H, S, HD), bf),
    )(sinks, qh, kh, vh)
    attn = oh.transpose(1, 0, 2).reshape(S, D)

    x2, t, i0, i1, w0, w1 = pl.pallas_call(
        _post_body,
        grid=(nS,),
        in_specs=[
            pl.BlockSpec((BS, D), lambda i: (i, 0)),
            pl.BlockSpec((BS, D), lambda i: (i, 0)),
            pl.BlockSpec((D, D), lambda i: (0, 0)),
            pl.BlockSpec((1, D), lambda i: (0, 0)),
            pl.BlockSpec((D, E), lambda i: (0, 0)),
            pl.BlockSpec((1, E), lambda i: (0, 0)),
        ],
        out_specs=[
            pl.BlockSpec((BS, D), lambda i: (i, 0)),
            pl.BlockSpec((BS, D), lambda i: (i, 0)),
            pl.BlockSpec((BS, 1), lambda i: (i, 0)),
            pl.BlockSpec((BS, 1), lambda i: (i, 0)),
            pl.BlockSpec((BS, 1), lambda i: (i, 0)),
            pl.BlockSpec((BS, 1), lambda i: (i, 0)),
        ],
        out_shape=[
            jax.ShapeDtypeStruct((S, D), jnp.float32),
            jax.ShapeDtypeStruct((S, D), bf),
            jax.ShapeDtypeStruct((S, 1), jnp.int32),
            jax.ShapeDtypeStruct((S, 1), jnp.int32),
            jax.ShapeDtypeStruct((S, 1), jnp.float32),
            jax.ShapeDtypeStruct((S, 1), jnp.float32),
        ],
    )(x, attn, wo_b, ln2_w.reshape(1, D), router_w, router_b.reshape(1, E))

    out = pl.pallas_call(
        _moe_body,
        grid=(nS, E),
        in_specs=[
            pl.BlockSpec((BS, D), lambda s, e: (s, 0)),
            pl.BlockSpec((BS, D), lambda s, e: (s, 0)),
            pl.BlockSpec((BS, 1), lambda s, e: (s, 0)),
            pl.BlockSpec((BS, 1), lambda s, e: (s, 0)),
            pl.BlockSpec((BS, 1), lambda s, e: (s, 0)),
            pl.BlockSpec((BS, 1), lambda s, e: (s, 0)),
            pl.BlockSpec((1, D, F), lambda s, e: (e, 0, 0)),
            pl.BlockSpec((1, 1, F), lambda s, e: (e, 0, 0)),
            pl.BlockSpec((1, D, F), lambda s, e: (e, 0, 0)),
            pl.BlockSpec((1, 1, F), lambda s, e: (e, 0, 0)),
            pl.BlockSpec((1, F, D), lambda s, e: (e, 0, 0)),
            pl.BlockSpec((1, 1, D), lambda s, e: (e, 0, 0)),
        ],
        out_specs=pl.BlockSpec((BS, D), lambda s, e: (s, 0)),
        out_shape=jax.ShapeDtypeStruct((S, D), jnp.float32),
        compiler_params=pltpu.CompilerParams(
            dimension_semantics=("parallel", "arbitrary")),
    )(x2, t, i0, i1, w0, w1, wg_b, b_gate.reshape(E, 1, F), wu_b,
      b_up.reshape(E, 1, F), wd_b, b_down.reshape(E, 1, D))

    return out.reshape(B, S, D)
